# baseline (speedup 1.0000x reference)
import functools

import jax
import jax.numpy as jnp
from jax import lax
from jax.experimental import pallas as pl
from jax.experimental.pallas import tpu as pltpu
from jax.experimental.pallas import tpu_sc as plsc

LANES = 16


def _build(B, S, D):
    NC, NS = 2, 16
    NW = NC * NS
    LB = B // NW
    SB = 4
    NBLK = S // SB
    D8 = D // 8
    TW = 129
    assert LB == 128 and S % (2 * SB) == 0 and D % LANES == 0

    mesh = plsc.VectorSubcoreMesh(core_axis_name="c", subcore_axis_name="s")

    @functools.partial(
        pl.kernel,
        out_type=jax.ShapeDtypeStruct((S, D8, NW, 8, 128), jnp.float32),
        mesh=mesh,
        compiler_params=pltpu.CompilerParams(use_tc_tiling_on_sc=False,
                                             needs_layout_passes=False),
        scratch_types=[
            pltpu.VMEM((S, D), jnp.float32),
            pltpu.VMEM((SB, LB), jnp.int32),
            pltpu.VMEM((SB, LB), jnp.int32),
            pltpu.VMEM((SB, LB, D), jnp.float32),
            pltpu.VMEM((SB, LB, D), jnp.float32),
            pltpu.VMEM((D, TW), jnp.float32),
            pltpu.VMEM((D, TW), jnp.float32),
            pltpu.SemaphoreType.DMA,
            pltpu.SemaphoreType.DMA,
            pltpu.SemaphoreType.DMA,
            pltpu.SemaphoreType.DMA,
        ],
    )
    def emb(xT_hbm, table_hbm, pos_hbm, out_hbm, pos_v,
            idx0, idx1, rin0, rin1, tb0, tb1,
            gsem0, gsem1, osem0, osem1):
        wid = lax.axis_index("s") * NC + lax.axis_index("c")
        pltpu.sync_copy(pos_hbm, pos_v)
        diota = [lax.iota(jnp.int32, LANES) + (k * LANES)
                 for k in range(D // LANES)]

        idxs = (idx0, idx1)
        rins = (rin0, rin1)
        tbs = (tb0, tb1)
        gsems = (gsem0, gsem1)
        osems = (osem0, osem1)

        def fire_gather(g, p):
            for j in range(SB):
                pltpu.sync_copy(xT_hbm.at[g * SB + j, wid], idxs[p].at[j])
            for j in range(SB):
                pltpu.async_copy(table_hbm.at[idxs[p].at[j]],
                                 rins[p].at[j], gsems[p])

        def wait_gather(p):
            for j in range(SB):
                pltpu.make_async_copy(table_hbm.at[idxs[p].at[j]],
                                      rins[p].at[j], gsems[p]).wait()

        def wait_out(q):
            for d8 in range(D8):
                pltpu.make_async_copy(
                    tbs[q].at[pl.ds(d8 * 8, 8), pl.ds(0, 128)],
                    out_hbm.at[0, d8, wid], osems[q]).wait()

        fire_gather(0, 0)

        def outer(g2, carry):
            for p in range(2):
                g = 2 * g2 + p

                @pl.when(g < NBLK - 1)
                def _():
                    fire_gather(g + 1, 1 - p)

                wait_gather(p)
                rin = rins[p]
                for j in range(SB):
                    s = g * SB + j
                    q = j % 2
                    tb = tbs[q]

                    @pl.when(s >= 2)
                    def _():
                        wait_out(q)

                    pos_regs = [pos_v[s, pl.ds(k * LANES, LANES)]
                                for k in range(D // LANES)]

                    @plsc.parallel_loop(0, LB, unroll=4)
                    def _(b):
                        bvec = lax.broadcast(b, (LANES,))
                        for k in range(D // LANES):
                            v = rin[j, b, pl.ds(k * LANES, LANES)] + pos_regs[k]
                            plsc.store_scatter(tb, [diota[k], bvec], v)

                    for d8 in range(D8):
                        pltpu.async_copy(
                            tb.at[pl.ds(d8 * 8, 8), pl.ds(0, 128)],
                            out_hbm.at[s, d8, wid], osems[q])
            return carry

        lax.fori_loop(0, NBLK // 2, outer, 0)
        wait_out(0)
        wait_out(1)

    return emb


def kernel(x, table, pos_table):
    B, S = x.shape
    D = table.shape[1]
    xT = lax.optimization_barrier(
        x.astype(jnp.int32).T.reshape(S, B // 128, 128))
    emb = _build(B, S, D)
    out5 = emb(xT, table, pos_table.astype(jnp.float32))
    return out5.transpose((2, 4, 0, 1, 3)).reshape(B, S, D)


# speedup vs baseline: 1.2331x; 1.2331x over previous
import functools

import jax
import jax.numpy as jnp
from jax import lax
from jax.experimental import pallas as pl
from jax.experimental.pallas import tpu as pltpu
from jax.experimental.pallas import tpu_sc as plsc

LANES = 16


def _build(B, S, D):
    NC, NS = 2, 16
    NW = NC * NS
    LB = B // NW
    SB = 4
    NBLK = S // SB
    D8 = D // 8
    TW = 129
    assert LB == 128 and S % (2 * SB) == 0 and D % LANES == 0

    mesh = plsc.VectorSubcoreMesh(core_axis_name="c", subcore_axis_name="s")

    @functools.partial(
        pl.kernel,
        out_type=jax.ShapeDtypeStruct((S, D8, NW, 8, 128), jnp.float32),
        mesh=mesh,
        compiler_params=pltpu.CompilerParams(use_tc_tiling_on_sc=False,
                                             needs_layout_passes=False),
        scratch_types=[
            pltpu.VMEM((S, D), jnp.float32),
            pltpu.VMEM((SB, LB), jnp.int32),
            pltpu.VMEM((SB, LB), jnp.int32),
            pltpu.VMEM((SB, LB, D), jnp.float32),
            pltpu.VMEM((SB, LB, D), jnp.float32),
            pltpu.VMEM((D, TW), jnp.float32),
            pltpu.VMEM((D, TW), jnp.float32),
            pltpu.SemaphoreType.DMA,
            pltpu.SemaphoreType.DMA,
            pltpu.SemaphoreType.DMA,
            pltpu.SemaphoreType.DMA,
        ],
    )
    def emb(xT_hbm, table_hbm, pos_hbm, out_hbm, pos_v,
            idx0, idx1, rin0, rin1, tb0, tb1,
            gsem0, gsem1, osem0, osem1):
        wid = lax.axis_index("s") * NC + lax.axis_index("c")
        pltpu.sync_copy(pos_hbm, pos_v)
        diota = [lax.iota(jnp.int32, LANES) + (k * LANES)
                 for k in range(D // LANES)]

        idxs = (idx0, idx1)
        rins = (rin0, rin1)
        tbs = (tb0, tb1)
        gsems = (gsem0, gsem1)
        osems = (osem0, osem1)

        def fire_gather(g, p):
            pltpu.sync_copy(xT_hbm.at[pl.ds(g * SB, SB), wid], idxs[p])
            for j in range(SB):
                pltpu.async_copy(table_hbm.at[idxs[p].at[j]],
                                 rins[p].at[j], gsems[p])

        def wait_gather(p):
            for j in range(SB):
                pltpu.make_async_copy(table_hbm.at[idxs[p].at[j]],
                                      rins[p].at[j], gsems[p]).wait()

        def wait_out(q):
            for d8 in range(D8):
                pltpu.make_async_copy(
                    tbs[q].at[pl.ds(d8 * 8, 8), pl.ds(0, 128)],
                    out_hbm.at[0, d8, wid], osems[q]).wait()

        fire_gather(0, 0)

        def outer(g2, carry):
            for p in range(2):
                g = 2 * g2 + p

                @pl.when(g < NBLK - 1)
                def _():
                    fire_gather(g + 1, 1 - p)

                wait_gather(p)
                rin = rins[p]
                for j in range(SB):
                    s = g * SB + j
                    q = j % 2
                    tb = tbs[q]

                    @pl.when(s >= 2)
                    def _():
                        wait_out(q)

                    pos_regs = [pos_v[s, pl.ds(k * LANES, LANES)]
                                for k in range(D // LANES)]

                    @plsc.parallel_loop(0, LB, unroll=4)
                    def _(b):
                        bvec = lax.broadcast(b, (LANES,))
                        for k in range(D // LANES):
                            v = rin[j, b, pl.ds(k * LANES, LANES)] + pos_regs[k]
                            plsc.store_scatter(tb, [diota[k], bvec], v)

                    for d8 in range(D8):
                        pltpu.async_copy(
                            tb.at[pl.ds(d8 * 8, 8), pl.ds(0, 128)],
                            out_hbm.at[s, d8, wid], osems[q])
            return carry

        lax.fori_loop(0, NBLK // 2, outer, 0)
        wait_out(0)
        wait_out(1)

    return emb


def kernel(x, table, pos_table):
    B, S = x.shape
    D = table.shape[1]
    xT = lax.optimization_barrier(
        x.astype(jnp.int32).T.reshape(S, B // 128, 128))
    emb = _build(B, S, D)
    out5 = emb(xT, table, pos_table.astype(jnp.float32))
    return out5.transpose((2, 4, 0, 1, 3)).reshape(B, S, D)


# speedup vs baseline: 1.2371x; 1.0032x over previous
import functools

import jax
import jax.numpy as jnp
from jax import lax
from jax.experimental import layout as jlayout
from jax.experimental import pallas as pl
from jax.experimental.pallas import tpu as pltpu
from jax.experimental.pallas import tpu_sc as plsc

LANES = 16


def _build(B, S, D):
    NC, NS = 2, 16
    NW = NC * NS
    LB = B // NW
    SB = 4
    NBLK = S // SB
    D8 = D // 8
    TW = 129
    assert LB == 128 and S % (2 * SB) == 0 and D % LANES == 0

    mesh = plsc.VectorSubcoreMesh(core_axis_name="c", subcore_axis_name="s")

    @functools.partial(
        pl.kernel,
        out_type=jax.ShapeDtypeStruct((S, D8, NW, 8, 128), jnp.float32),
        mesh=mesh,
        compiler_params=pltpu.CompilerParams(use_tc_tiling_on_sc=False,
                                             needs_layout_passes=False),
        scratch_types=[
            pltpu.VMEM((S, D), jnp.float32),
            pltpu.VMEM((SB, LB), jnp.int32),
            pltpu.VMEM((SB, LB), jnp.int32),
            pltpu.VMEM((SB, LB, D), jnp.float32),
            pltpu.VMEM((SB, LB, D), jnp.float32),
            pltpu.VMEM((D, TW), jnp.float32),
            pltpu.VMEM((D, TW), jnp.float32),
            pltpu.SemaphoreType.DMA,
            pltpu.SemaphoreType.DMA,
            pltpu.SemaphoreType.DMA,
            pltpu.SemaphoreType.DMA,
        ],
    )
    def emb(xT_hbm, table_hbm, pos_hbm, out_hbm, pos_v,
            idx0, idx1, rin0, rin1, tb0, tb1,
            gsem0, gsem1, osem0, osem1):
        wid = lax.axis_index("s") * NC + lax.axis_index("c")
        pltpu.sync_copy(pos_hbm, pos_v)
        diota = [lax.iota(jnp.int32, LANES) + (k * LANES)
                 for k in range(D // LANES)]

        idxs = (idx0, idx1)
        rins = (rin0, rin1)
        tbs = (tb0, tb1)
        gsems = (gsem0, gsem1)
        osems = (osem0, osem1)

        def fire_gather(g, p):
            pltpu.sync_copy(xT_hbm.at[pl.ds(g * SB, SB), wid], idxs[p])
            for j in range(SB):
                pltpu.async_copy(table_hbm.at[idxs[p].at[j]],
                                 rins[p].at[j], gsems[p])

        def wait_gather(p):
            for j in range(SB):
                pltpu.make_async_copy(table_hbm.at[idxs[p].at[j]],
                                      rins[p].at[j], gsems[p]).wait()

        def wait_out(q):
            for d8 in range(D8):
                pltpu.make_async_copy(
                    tbs[q].at[pl.ds(d8 * 8, 8), pl.ds(0, 128)],
                    out_hbm.at[0, d8, wid], osems[q]).wait()

        fire_gather(0, 0)

        def outer(g2, carry):
            for p in range(2):
                g = 2 * g2 + p

                @pl.when(g < NBLK - 1)
                def _():
                    fire_gather(g + 1, 1 - p)

                wait_gather(p)
                rin = rins[p]
                for j in range(SB):
                    s = g * SB + j
                    q = j % 2
                    tb = tbs[q]

                    @pl.when(s >= 2)
                    def _():
                        wait_out(q)

                    pos_regs = [pos_v[s, pl.ds(k * LANES, LANES)]
                                for k in range(D // LANES)]

                    @plsc.parallel_loop(0, LB, unroll=4)
                    def _(b):
                        bvec = lax.broadcast(b, (LANES,))
                        for k in range(D // LANES):
                            v = rin[j, b, pl.ds(k * LANES, LANES)] + pos_regs[k]
                            plsc.store_scatter(tb, [diota[k], bvec], v)

                    for d8 in range(D8):
                        pltpu.async_copy(
                            tb.at[pl.ds(d8 * 8, 8), pl.ds(0, 128)],
                            out_hbm.at[s, d8, wid], osems[q])
            return carry

        lax.fori_loop(0, NBLK // 2, outer, 0)
        wait_out(0)
        wait_out(1)

    return emb


def kernel(x, table, pos_table):
    B, S = x.shape
    D = table.shape[1]
    xT = jlayout.with_layout_constraint(
        x.astype(jnp.int32).T.reshape(S, B // 128, 128),
        jlayout.Layout(major_to_minor=(0, 1, 2), tiling=()))
    emb = _build(B, S, D)
    out5 = emb(xT, table, pos_table.astype(jnp.float32))
    return out5.transpose((2, 4, 0, 1, 3)).reshape(B, S, D)
